# Initial kernel scaffold; baseline (speedup 1.0000x reference)
import jax
import jax.numpy as jnp
from jax.experimental import pallas as pl


def kernel(images, predictions):
    raise NotImplementedError("write your pallas kernel here")


# speedup vs baseline: 34.7563x; 34.7563x over previous
import numpy as np
import jax
import jax.numpy as jnp
from jax import lax
from jax.experimental import pallas as pl
from jax.experimental.pallas import tpu as pltpu
from jax.experimental.pallas import tpu_sc as plsc

B = 2
A = 49104
C = 80
NC, NS, L = 2, 16, 16
CPW = C // NS
NV = A // L
GRP = 9
NG = NV // GRP
BMV = 31
KMAX = 100
SLOTS = 128
PSLOT = 144
CAP = 8192
MCAP = C * SLOTS
ROWW = 88
INT_MIN = -(2 ** 31)
INT_MAX = 2 ** 31 - 1
FNEG = -3.0e38
CONF = 0.05


def _np_anchors():
    ratios = [0.5, 1.0, 2.0]
    scales = [2.0 ** 0.0, 2.0 ** (1.0 / 3.0), 2.0 ** (2.0 / 3.0)]
    areas = [32.0 ** 2, 64.0 ** 2, 128.0 ** 2, 256.0 ** 2, 512.0 ** 2]
    out = []
    for i, level in enumerate(range(3, 8)):
        dims = []
        for r in ratios:
            ah = np.sqrt(areas[i] / r)
            aw = areas[i] / ah
            for s in scales:
                dims.append([s * aw, s * ah])
        dims = np.array(dims, dtype=np.float32)
        stride = 2 ** level
        fh = int(np.ceil(512 / stride))
        fw = int(np.ceil(512 / stride))
        rx = (np.arange(fw, dtype=np.float32) + 0.5) * stride
        ry = (np.arange(fh, dtype=np.float32) + 0.5) * stride
        cx, cy = np.meshgrid(rx, ry)
        centers = np.stack([cx, cy], axis=-1)
        centers = np.tile(centers[:, :, None, :], [1, 1, 9, 1])
        d = np.broadcast_to(dims[None, None], (fh, fw, 9, 2))
        out.append(np.concatenate([centers, d], axis=-1).reshape(-1, 4))
    anc = np.concatenate(out, axis=0).astype(np.float32)
    return np.pad(anc, ((0, 0), (0, 12)))


_ANCHORS = _np_anchors()


def _ln():
    return lax.iota(jnp.int32, L)


def _red(v, op):
    ln = _ln()
    for sh in (8, 4, 2, 1):
        v = op(v, v[(ln + sh) % L])
    return v


def _lmax(v):
    return _red(v, jnp.maximum)


def _lmin(v):
    return _red(v, jnp.minimum)


def _lsum(v):
    return _red(v, jnp.add)


def _sload(ref, i):
    return ref[pl.ds(i, L)][0]


def _pick(ref, base, i):
    return ref[pl.ds(base + i, L)][0]


def _srmw(ref, i, val):
    old = ref[pl.ds(i, L)]
    ref[pl.ds(i, L)] = jnp.where(_ln() == 0, val, old)


def _sc_body(keys_hbm, preds_hbm, anch_hbm,
             out_box, out_sc, out_cls, out_valid,
             col, bm, bm2, ck, ci, msum,
             tk, tg, ta, prow, arow,
             bx1, by1, bx2, by2, bar, bsc, bkp,
             stg_sc, stg_box,
             m_sc, mck, mci, mms, tmp16,
             o_sc, o_cls, obox, o_valid,
             s_sc, s_box, sem, sem2):
    b = lax.axis_index("c")
    s = lax.axis_index("s")
    ln = _ln()
    zl = ln * 0

    for v in range(PSLOT // L):
        tk[pl.ds(v * L, L)] = zl + INT_MIN
        tg[pl.ds(v * L, L)] = zl
        ta[pl.ds(v * L, L)] = zl

    def run_problem(q, _):
        c = s * CPW + q
        base = (b * C + c) * A
        pltpu.sync_copy(keys_hbm.at[pl.ds(base, A)], col)

        def p1(g, _2):
            vs = [col[pl.ds(g * (GRP * L) + u * L, L)] for u in range(GRP)]
            m = vs[0]
            for u in range(1, GRP):
                m = jnp.maximum(m, vs[u])
            bm[pl.ds(g * L, L)] = m
            return 0

        lax.fori_loop(0, NG, p1, 0)

        def p1b(k, _2):
            m = bm[pl.ds(k * (11 * L), L)]
            for u in range(1, 11):
                m = jnp.maximum(m, bm[pl.ds(k * (11 * L) + u * L, L)])
            bm2[pl.ds(k * L, L)] = m
            return 0

        lax.fori_loop(0, BMV, p1b, 0)

        def bis(_2, lohi):
            lo, hi = lohi
            mid = (lo >> 1) + (hi >> 1) + (lo & hi & 1)
            acc = zl
            for k in range(BMV):
                acc = acc + jnp.where(bm2[pl.ds(k * L, L)] > mid, 1, 0)
            cnt = _lsum(acc)[0]
            ok = cnt >= KMAX
            return (jnp.where(ok, mid, lo), jnp.where(ok, hi, mid))

        t, _2 = lax.fori_loop(0, 32, bis,
                              (jnp.int32(INT_MIN), jnp.int32(INT_MAX)))

        def p2(g, nc):
            gm = bm[pl.ds(g * L, L)]
            hit = _lmax(gm)[0] > t

            def dohit(nc2):
                for u in range(GRP):
                    off = g * (GRP * L) + u * L
                    v = col[pl.ds(off, L)]
                    hv = _lmax(v)[0] > t

                    def app(nc3):
                        ck[pl.ds(nc3, L)] = jnp.where(v > t, v,
                                                      jnp.int32(INT_MIN))
                        ci[pl.ds(nc3, L)] = zl + off + ln
                        return jnp.minimum(nc3 + L, jnp.int32(CAP - L))

                    nc2 = lax.cond(hv, app, lambda n: n, nc2)
                return nc2

            return lax.cond(hit, dohit, lambda n: n, nc)

        nc = lax.fori_loop(0, NG, p2, jnp.int32(0))
        nmv = nc >> 4

        for v in range(CAP // L // L):
            msum[pl.ds(v * L, L)] = zl + INT_MIN

        def mb(k, _2):
            lm = _lmax(ck[pl.ds(k * L, L)])[0]
            off = (k >> 4) * L
            mv = msum[pl.ds(off, L)]
            msum[pl.ds(off, L)] = jnp.where(ln == (k & 15), lm, mv)
            return 0

        lax.fori_loop(0, nmv, mb, 0)

        def ex(i, _2):
            def sweep(k, carry):
                mvec, kvec = carry
                cur = msum[pl.ds(k * L, L)]
                upd = cur > mvec
                return jnp.maximum(mvec, cur), jnp.where(upd, k, kvec)

            nms_ = (nmv + 15) >> 4
            mvec, kvec = lax.fori_loop(0, nms_, sweep,
                                       (zl + INT_MIN, zl))
            m = _lmax(mvec)[0]
            pos = _lmin(jnp.where(mvec == m, kvec * L + ln,
                                  jnp.int32(INT_MAX)))[0]
            pos = jnp.minimum(pos, jnp.int32(CAP // L - 1))
            kk = pos
            vv = ck[pl.ds(kk * L, L)]
            lpos = _lmin(jnp.where(vv == m, ln, jnp.int32(L)))[0]
            lpos = jnp.minimum(lpos, jnp.int32(L - 1))
            aidx = _pick(ci, kk * L, lpos)
            _srmw(tk, i, m)
            _srmw(ta, i, aidx)
            _srmw(tg, i, b * A + aidx)
            ck[pl.ds(kk * L, L)] = jnp.where(ln == lpos, jnp.int32(INT_MIN),
                                             vv)
            lm2 = _lmax(ck[pl.ds(kk * L, L)])[0]
            off = (kk >> 4) * L
            mv = msum[pl.ds(off, L)]
            msum[pl.ds(off, L)] = jnp.where(ln == (kk & 15), lm2, mv)
            return 0

        lax.fori_loop(0, KMAX, ex, 0)

        for r in range(SLOTS // L):
            for jj in range(L):
                i = r * L + jj
                g = _sload(tg, i)
                a = _sload(ta, i)
                pltpu.async_copy(preds_hbm.at[pl.ds(g * ROWW, ROWW)],
                                 prow.at[pl.ds(i * ROWW, ROWW)], sem)
                pltpu.async_copy(anch_hbm.at[pl.ds(a * L, L)],
                                 arow.at[pl.ds(i * L, L)], sem2)
            pltpu.make_async_copy(
                preds_hbm.at[pl.ds(0, L * ROWW)],
                prow.at[pl.ds(r * L * ROWW, L * ROWW)], sem).wait()
            pltpu.make_async_copy(
                anch_hbm.at[pl.ds(0, L * L)],
                arow.at[pl.ds(r * L * L, L * L)], sem2).wait()

        for v in range(SLOTS // L):
            dx = zl * 0.0
            dy = dx
            dw = dx
            dh = dx
            ax = dx
            ay = dx
            aw = dx
            ah = dx
            lg = dx
            for jj in range(L):
                i = v * L + jj
                pr = prow[pl.ds(i * ROWW, L)]
                ar = arow[pl.ds(i * L, L)]
                lgv = prow[pl.ds(i * ROWW + 4 + c, L)][0]
                sel = ln == jj
                dx = jnp.where(sel, pr[0], dx)
                dy = jnp.where(sel, pr[1], dy)
                dw = jnp.where(sel, pr[2], dw)
                dh = jnp.where(sel, pr[3], dh)
                ax = jnp.where(sel, ar[0], ax)
                ay = jnp.where(sel, ar[1], ay)
                aw = jnp.where(sel, ar[2], aw)
                ah = jnp.where(sel, ar[3], ah)
                lg = jnp.where(sel, lgv, lg)
            cx = dx * 0.1 * aw + ax
            cy = dy * 0.1 * ah + ay
            w = jnp.exp(dw * 0.2) * aw
            h = jnp.exp(dh * 0.2) * ah
            sl = pl.ds(v * L, L)
            bx1[sl] = cx - w * 0.5
            by1[sl] = cy - h * 0.5
            bx2[sl] = cx + w * 0.5
            by2[sl] = cy + h * 0.5
            bar[sl] = w * h
            sc = 1.0 / (1.0 + jnp.exp(-lg))
            bsc[sl] = sc
            bkp[sl] = jnp.where((sc >= CONF) & ((ln + v * L) < KMAX), 1, 0)

        def nms(i, _2):
            @pl.when(_sload(bkp, i) != 0)
            def _3():
                x1i = _sload(bx1, i)
                y1i = _sload(by1, i)
                x2i = _sload(bx2, i)
                y2i = _sload(by2, i)
                ai = _sload(bar, i)
                for v in range(SLOTS // L):
                    sl = pl.ds(v * L, L)
                    jv = ln + v * L
                    ww = jnp.maximum(
                        jnp.minimum(bx2[sl], x2i) - jnp.maximum(bx1[sl], x1i),
                        0.0)
                    hh = jnp.maximum(
                        jnp.minimum(by2[sl], y2i) - jnp.maximum(by1[sl], y1i),
                        0.0)
                    inter = ww * hh
                    union = bar[sl] + ai - inter
                    sup = (inter + inter > jnp.maximum(union, 1e-8)) & (jv > i)
                    bkp[sl] = jnp.where(sup, 0, bkp[sl])
            return 0

        lax.fori_loop(0, KMAX, nms, 0)

        for v in range(SLOTS // L):
            sl = pl.ds(v * L, L)
            stg_sc[sl] = jnp.where(bkp[sl] != 0, bsc[sl], -1.0)
        for j in range(SLOTS // 2):
            av = _sload(bx1, 2 * j)
            bv = _sload(by1, 2 * j)
            cv = _sload(bx2, 2 * j)
            dv = _sload(by2, 2 * j)
            a2 = _sload(bx1, 2 * j + 1)
            b2 = _sload(by1, 2 * j + 1)
            c2 = _sload(bx2, 2 * j + 1)
            d2 = _sload(by2, 2 * j + 1)
            row = jnp.where(ln == 0, av, 0.0)
            row = jnp.where(ln == 1, bv, row)
            row = jnp.where(ln == 2, cv, row)
            row = jnp.where(ln == 3, dv, row)
            row = jnp.where(ln == 8, a2, row)
            row = jnp.where(ln == 9, b2, row)
            row = jnp.where(ln == 10, c2, row)
            row = jnp.where(ln == 11, d2, row)
            stg_box[pl.ds(j * L, L)] = row
        pltpu.sync_copy(stg_sc, s_sc.at[pl.ds(c * SLOTS, SLOTS)])
        pltpu.sync_copy(stg_box, s_box.at[pl.ds(c * SLOTS * 8, SLOTS * 8)])
        return 0

    lax.fori_loop(0, CPW, run_problem, 0)
    plsc.subcore_barrier()

    @pl.when(s == 0)
    def _m():
        pltpu.sync_copy(s_sc, m_sc)
        for v in range(SLOTS // L):
            o_sc[pl.ds(v * L, L)] = zl * 0.0
            o_cls[pl.ds(v * L, L)] = zl * 0.0
        for v in range(SLOTS // 2):
            obox[pl.ds(v * L, L)] = zl * 0.0

        def mscan(cc, nc):
            vs = [m_sc[pl.ds(cc * SLOTS + u * L, L)]
                  for u in range(SLOTS // L)]
            gm = vs[0]
            for u in range(1, SLOTS // L):
                gm = jnp.maximum(gm, vs[u])
            hit = _lmax(gm)[0] > 0.0

            def dohit(nc2):
                for u in range(SLOTS // L):
                    v = vs[u]
                    hv = _lmax(v)[0] > 0.0

                    def app(nc3):
                        mck[pl.ds(nc3, L)] = jnp.where(v > 0.0, v, FNEG)
                        mci[pl.ds(nc3, L)] = zl + cc * SLOTS + u * L + ln
                        return jnp.minimum(nc3 + L, jnp.int32(MCAP))

                    nc2 = lax.cond(hv, app, lambda n: n, nc2)
                return nc2

            return lax.cond(hit, dohit, lambda n: n, nc)

        nc = lax.fori_loop(0, C, mscan, jnp.int32(0))
        nmv = nc >> 4

        for v in range((MCAP // L + L - 1) // L):
            mms[pl.ds(v * L, L)] = zl * 0.0 + FNEG

        def mb2(k, _2):
            lm = _lmax(mck[pl.ds(k * L, L)])[0]
            off = (k >> 4) * L
            mv = mms[pl.ds(off, L)]
            mms[pl.ds(off, L)] = jnp.where(ln == (k & 15), lm, mv)
            return 0

        lax.fori_loop(0, nmv, mb2, 0)

        def mex(i, nval):
            def sweep(k, carry):
                mvec, kvec = carry
                cur = mms[pl.ds(k * L, L)]
                upd = cur > mvec
                return jnp.maximum(mvec, cur), jnp.where(upd, k, kvec)

            nms_ = (nmv + 15) >> 4
            mvec, kvec = lax.fori_loop(0, nms_, sweep,
                                       (zl * 0.0 + FNEG, zl))
            m = _lmax(mvec)[0]
            alive = m > 0.0
            pos = _lmin(jnp.where(mvec == m, kvec * L + ln,
                                  jnp.int32(INT_MAX)))[0]
            kk = jnp.minimum(pos, jnp.int32(MCAP // L - 1))
            vv = mck[pl.ds(kk * L, L)]
            lpos = _lmin(jnp.where(vv == m, ln, jnp.int32(L)))[0]
            lpos = jnp.minimum(lpos, jnp.int32(L - 1))
            fp = _pick(mci, kk * L, lpos)
            mck[pl.ds(kk * L, L)] = jnp.where(ln == lpos,
                                              jnp.float32(FNEG), vv)
            lm2 = _lmax(mck[pl.ds(kk * L, L)])[0]
            off = (kk >> 4) * L
            mv = mms[pl.ds(off, L)]
            mms[pl.ds(off, L)] = jnp.where(ln == (kk & 15), lm2, mv)

            _srmw(o_sc, i, jnp.where(alive, m, 0.0))
            _srmw(o_cls, i,
                  jnp.where(alive, (fp >> 7).astype(jnp.float32), 0.0))

            @pl.when(alive)
            def _4():
                pltpu.sync_copy(s_box.at[pl.ds(fp * 8, L)], tmp16)
                bxv = tmp16[pl.ds(0, L)]
                old = obox[pl.ds(i * 8, L)]
                obox[pl.ds(i * 8, L)] = jnp.where(
                    ln < 4, bxv, jnp.where(ln < 8, 0.0, old))

            return nval + jnp.where(alive, 1, 0)

        nvalid = lax.fori_loop(0, KMAX, mex, jnp.int32(0))
        o_valid[pl.ds(0, L)] = zl + nvalid

        pltpu.sync_copy(obox, out_box.at[b])
        pltpu.sync_copy(o_sc, out_sc.at[b])
        pltpu.sync_copy(o_cls, out_cls.at[b])
        pltpu.sync_copy(o_valid, out_valid.at[b])


def kernel(images, predictions):
    del images
    logits = predictions[:, :, 4:]
    bits = lax.bitcast_convert_type(logits, jnp.int32)
    keys = jnp.where(bits < 0, bits ^ jnp.int32(0x7FFFFFFF), bits)
    keys_flat = jnp.transpose(keys, (0, 2, 1)).reshape(-1)
    preds_flat = jnp.pad(predictions.reshape(B * A, 84),
                         ((0, 0), (0, 4))).reshape(-1)
    anchors_flat = jnp.asarray(_ANCHORS.reshape(-1))

    mesh = plsc.VectorSubcoreMesh(core_axis_name="c", subcore_axis_name="s",
                                  num_cores=NC, num_subcores=NS)
    f32, i32 = jnp.float32, jnp.int32
    run = pl.kernel(
        _sc_body,
        compiler_params=pltpu.CompilerParams(use_tc_tiling_on_sc=False),
        out_type=(
            jax.ShapeDtypeStruct((B, SLOTS * 8), f32),
            jax.ShapeDtypeStruct((B, SLOTS), f32),
            jax.ShapeDtypeStruct((B, SLOTS), f32),
            jax.ShapeDtypeStruct((B, L), i32),
        ),
        mesh=mesh,
        scratch_types=(
            pltpu.VMEM((A,), i32),
            pltpu.VMEM((NG * L,), i32),
            pltpu.VMEM((BMV * L,), i32),
            pltpu.VMEM((CAP,), i32),
            pltpu.VMEM((CAP + L,), i32),
            pltpu.VMEM((CAP // L,), i32),
            pltpu.VMEM((PSLOT,), i32),
            pltpu.VMEM((PSLOT,), i32),
            pltpu.VMEM((PSLOT,), i32),
            pltpu.VMEM((SLOTS * ROWW + L,), f32),
            pltpu.VMEM((SLOTS * L,), f32),
            pltpu.VMEM((PSLOT,), f32),
            pltpu.VMEM((PSLOT,), f32),
            pltpu.VMEM((PSLOT,), f32),
            pltpu.VMEM((PSLOT,), f32),
            pltpu.VMEM((PSLOT,), f32),
            pltpu.VMEM((PSLOT,), f32),
            pltpu.VMEM((PSLOT,), i32),
            pltpu.VMEM((SLOTS,), f32),
            pltpu.VMEM((SLOTS * 8,), f32),
            pltpu.VMEM((C * SLOTS,), f32),
            pltpu.VMEM((MCAP + L,), f32),
            pltpu.VMEM((MCAP + L,), i32),
            pltpu.VMEM((MCAP // L,), f32),
            pltpu.VMEM((L,), f32),
            pltpu.VMEM((SLOTS,), f32),
            pltpu.VMEM((SLOTS,), f32),
            pltpu.VMEM((SLOTS * 8,), f32),
            pltpu.VMEM((L,), i32),
            pltpu.VMEM_SHARED((C * SLOTS,), f32),
            pltpu.VMEM_SHARED((C * SLOTS * 8 + L,), f32),
            pltpu.SemaphoreType.DMA,
            pltpu.SemaphoreType.DMA,
        ),
    )
    boxes_p, scores_p, classes_p, valid_p = run(keys_flat, preds_flat,
                                                anchors_flat)
    boxes = boxes_p.reshape(B, SLOTS, 8)[:, :KMAX, :4]
    return (boxes, scores_p[:, :KMAX], classes_p[:, :KMAX], valid_p[:, 0])
